# baseline (speedup 1.0000x reference)
import functools

import jax
import jax.numpy as jnp
import numpy as np
from jax import lax
from jax.experimental import pallas as pl
from jax.experimental.pallas import tpu as pltpu
from jax.experimental.pallas import tpu_sc as plsc

N_NODES = 10000
N_NETS = 2000
E = 160000
D_NODE = 128
D_NET = 16
D_PIN = 16
D_OUT_NODE = 16
D_OUT_NET = 128

NC = 2
NS = 16
CHUNK = 128
N_CH = 40
GRP = 4
E_PT = CHUNK * N_CH
E_PAD = E_PT * NC * NS
N_NODES_P = 10240
N_NETS_P = 2048

_MESH = plsc.VectorSubcoreMesh(core_axis_name="c", subcore_axis_name="s")
_SC_PARAMS = pltpu.CompilerParams(use_tc_tiling_on_sc=False)


def _sc1_body(ps_cnt, pd_cnt, nd_cnt, nsrc_idx, net_feat, ones16, zc,
              cnt_src, cnt_dst, cnt_in, srcnet,
              idx_a, idx_b, idx_c, idx_d, ones_v, rows_v, gsem, ssem,
              cs_sh, cd_sh, ci_sh):
  c = lax.axis_index("c")
  s = lax.axis_index("s")
  tbase = (c * NS + s) * N_CH

  r0 = s * (N_NODES_P // NS)
  pltpu.sync_copy(zc.at[pl.ds(r0, N_NODES_P // NS)],
                  cs_sh.at[pl.ds(r0, N_NODES_P // NS)])
  pltpu.sync_copy(zc.at[pl.ds(r0, N_NODES_P // NS)],
                  ci_sh.at[pl.ds(r0, N_NODES_P // NS)])
  r1 = s * (N_NETS_P // NS)
  pltpu.sync_copy(zc.at[pl.ds(r1, N_NETS_P // NS)],
                  cd_sh.at[pl.ds(r1, N_NETS_P // NS)])

  pltpu.sync_copy(ones16, ones_v)
  pltpu.sync_copy(ps_cnt.at[pl.ds(tbase, N_CH)], idx_a)
  pltpu.sync_copy(pd_cnt.at[pl.ds(tbase, N_CH)], idx_b)
  pltpu.sync_copy(nd_cnt.at[pl.ds(tbase, N_CH)], idx_c)
  pltpu.sync_copy(nsrc_idx.at[pl.ds(tbase, N_CH)], idx_d)
  plsc.subcore_barrier()

  ones_bytes = CHUNK * 16 * 4

  def count_into(idx_all, sh):
    def fire(j, carry):
      pltpu.async_copy(ones_v, sh.at[idx_all.at[j]], ssem, add=True)
      return carry
    lax.fori_loop(0, N_CH, fire, 0)
    def drain(j, carry):
      pltpu.make_async_copy(ones16, ones_v, ssem).wait()
      return carry
    lax.fori_loop(0, N_CH, drain, 0)

  count_into(idx_a, cs_sh)
  count_into(idx_b, cd_sh)
  count_into(idx_c, ci_sh)

  def ggroup(g, carry):
    j0 = g * GRP
    for b in range(GRP):
      pltpu.async_copy(net_feat.at[idx_d.at[j0 + b]],
                       rows_v.at[b], gsem)
    for b in range(GRP):
      pltpu.make_async_copy(net_feat.at[idx_d.at[j0 + b]],
                            rows_v.at[b], gsem).wait()
      pltpu.async_copy(rows_v.at[b],
                       srcnet.at[pl.ds((tbase + j0 + b) * CHUNK, CHUNK)],
                       ssem)
    for b in range(GRP):
      pltpu.make_async_copy(rows_v.at[b],
                            srcnet.at[pl.ds(0, CHUNK)], ssem).wait()
    return carry
  lax.fori_loop(0, N_CH // GRP, ggroup, 0)

  plsc.subcore_barrier()

  pltpu.sync_copy(cs_sh.at[pl.ds(r0, N_NODES_P // NS)],
                  cnt_src.at[c, pl.ds(r0, N_NODES_P // NS)])
  pltpu.sync_copy(ci_sh.at[pl.ds(r0, N_NODES_P // NS)],
                  cnt_in.at[c, pl.ds(r0, N_NODES_P // NS)])
  pltpu.sync_copy(cd_sh.at[pl.ds(r1, N_NETS_P // NS)],
                  cnt_dst.at[c, pl.ds(r1, N_NETS_P // NS)])


_sc1 = functools.partial(
    pl.kernel,
    mesh=_MESH,
    compiler_params=_SC_PARAMS,
    out_type=[
        jax.ShapeDtypeStruct((NC, N_NODES_P, 16), jnp.float32),
        jax.ShapeDtypeStruct((NC, N_NETS_P, 16), jnp.float32),
        jax.ShapeDtypeStruct((NC, N_NODES_P, 16), jnp.float32),
        jax.ShapeDtypeStruct((E_PAD, D_NET), jnp.float32),
    ],
    scratch_types=[
        pltpu.VMEM((N_CH, CHUNK), jnp.int32),
        pltpu.VMEM((N_CH, CHUNK), jnp.int32),
        pltpu.VMEM((N_CH, CHUNK), jnp.int32),
        pltpu.VMEM((N_CH, CHUNK), jnp.int32),
        pltpu.VMEM((CHUNK, 16), jnp.float32),
        pltpu.VMEM((GRP, CHUNK, D_NET), jnp.float32),
        pltpu.SemaphoreType.DMA,
        pltpu.SemaphoreType.DMA,
        pltpu.VMEM_SHARED((N_NODES_P, 16), jnp.float32),
        pltpu.VMEM_SHARED((N_NETS_P, 16), jnp.float32),
        pltpu.VMEM_SHARED((N_NODES_P, 16), jnp.float32),
    ],
)(_sc1_body)


def _sc2_body(h, m, ps_gat, pd_idx, nd_idx, zbig, zsmall,
              agg, nacc,
              idx_s, idx_p, idx_n, hrows, mrows, gsem, msem, ssem,
              agg_sh, nacc_sh):
  c = lax.axis_index("c")
  s = lax.axis_index("s")
  tbase = (c * NS + s) * N_CH

  r0 = s * (N_NETS_P // NS)
  pltpu.sync_copy(zbig.at[pl.ds(r0, N_NETS_P // NS)],
                  agg_sh.at[pl.ds(r0, N_NETS_P // NS)])
  r1 = s * (N_NODES_P // NS)
  pltpu.sync_copy(zsmall.at[pl.ds(r1, N_NODES_P // NS)],
                  nacc_sh.at[pl.ds(r1, N_NODES_P // NS)])

  pltpu.sync_copy(ps_gat.at[pl.ds(tbase, N_CH)], idx_s)
  pltpu.sync_copy(pd_idx.at[pl.ds(tbase, N_CH)], idx_p)
  pltpu.sync_copy(nd_idx.at[pl.ds(tbase, N_CH)], idx_n)
  plsc.subcore_barrier()

  def group(g, carry):
    j0 = g * GRP
    for b in range(GRP):
      pltpu.async_copy(h.at[idx_s.at[j0 + b]], hrows.at[b], gsem)
      pltpu.async_copy(m.at[pl.ds((tbase + j0 + b) * CHUNK, CHUNK)],
                       mrows.at[b], msem)
    for b in range(GRP):
      pltpu.make_async_copy(h.at[idx_s.at[j0 + b]], hrows.at[b], gsem).wait()
      pltpu.async_copy(hrows.at[b], agg_sh.at[idx_p.at[j0 + b]], ssem,
                       add=True)
    for b in range(GRP):
      pltpu.make_async_copy(m.at[pl.ds(0, CHUNK)], mrows.at[b], msem).wait()
      pltpu.async_copy(mrows.at[b], nacc_sh.at[idx_n.at[j0 + b]], ssem,
                       add=True)
    for b in range(GRP):
      pltpu.make_async_copy(hrows.at[b], agg_sh.at[pl.ds(0, CHUNK)],
                            ssem).wait()
      pltpu.make_async_copy(mrows.at[b], nacc_sh.at[pl.ds(0, CHUNK)],
                            ssem).wait()
    return carry
  lax.fori_loop(0, N_CH // GRP, group, 0)

  plsc.subcore_barrier()

  pltpu.sync_copy(agg_sh.at[pl.ds(r0, N_NETS_P // NS)],
                  agg.at[c, pl.ds(r0, N_NETS_P // NS)])
  pltpu.sync_copy(nacc_sh.at[pl.ds(r1, N_NODES_P // NS)],
                  nacc.at[c, pl.ds(r1, N_NODES_P // NS)])


_sc2 = functools.partial(
    pl.kernel,
    mesh=_MESH,
    compiler_params=_SC_PARAMS,
    out_type=[
        jax.ShapeDtypeStruct((NC, N_NETS_P, D_OUT_NET), jnp.float32),
        jax.ShapeDtypeStruct((NC, N_NODES_P, D_OUT_NODE), jnp.float32),
    ],
    scratch_types=[
        pltpu.VMEM((N_CH, CHUNK), jnp.int32),
        pltpu.VMEM((N_CH, CHUNK), jnp.int32),
        pltpu.VMEM((N_CH, CHUNK), jnp.int32),
        pltpu.VMEM((GRP, CHUNK, D_OUT_NET), jnp.float32),
        pltpu.VMEM((GRP, CHUNK, D_OUT_NODE), jnp.float32),
        pltpu.SemaphoreType.DMA,
        pltpu.SemaphoreType.DMA,
        pltpu.SemaphoreType.DMA,
        pltpu.VMEM_SHARED((N_NETS_P, D_OUT_NET), jnp.float32),
        pltpu.VMEM_SHARED((N_NODES_P, D_OUT_NODE), jnp.float32),
    ],
)(_sc2_body)


_H_BLK = 1000


def _h_body(x_ref, c0_ref, c1_ref, w_ref, o_ref):
  cnt = c0_ref[...][:, :1] + c1_ref[...][:, :1]
  scale = lax.rsqrt(jnp.maximum(cnt, 1.0))
  o_ref[...] = jnp.dot(x_ref[...] * scale, w_ref[...],
                       preferred_element_type=jnp.float32)


def _h_call(node_feat, c0, c1, w):
  grid = N_NODES // _H_BLK
  return pl.pallas_call(
      _h_body,
      grid=(grid,),
      in_specs=[
          pl.BlockSpec((_H_BLK, D_NODE), lambda i: (i, 0)),
          pl.BlockSpec((_H_BLK, 16), lambda i: (i, 0)),
          pl.BlockSpec((_H_BLK, 16), lambda i: (i, 0)),
          pl.BlockSpec((D_NODE, D_OUT_NET), lambda i: (0, 0)),
      ],
      out_specs=pl.BlockSpec((_H_BLK, D_OUT_NET), lambda i: (i, 0)),
      out_shape=jax.ShapeDtypeStruct((N_NODES, D_OUT_NET), jnp.float32),
  )(node_feat, c0, c1, w)


_M_BLK = 2048


def _m_body(pin_ref, sn_ref, r_ref, s_ref, t2_ref, b_ref, o_ref):
  pin = pin_ref[...]
  sn = sn_ref[...]
  zr = jnp.dot(pin, r_ref[...], preferred_element_type=jnp.float32)
  zt = jnp.dot(sn, s_ref[...], preferred_element_type=jnp.float32)
  o_ref[...] = (jnp.dot(zr * zt, t2_ref[...], preferred_element_type=jnp.float32)
                + jnp.dot(sn, b_ref[...], preferred_element_type=jnp.float32))


def _m_call(pin_feat, srcnet, rmat, smat, t2, bmat):
  grid = E_PAD // _M_BLK
  return pl.pallas_call(
      _m_body,
      grid=(grid,),
      in_specs=[
          pl.BlockSpec((_M_BLK, D_PIN), lambda i: (i, 0)),
          pl.BlockSpec((_M_BLK, D_NET), lambda i: (i, 0)),
          pl.BlockSpec((D_PIN, D_PIN * D_NET), lambda i: (0, 0)),
          pl.BlockSpec((D_NET, D_PIN * D_NET), lambda i: (0, 0)),
          pl.BlockSpec((D_PIN * D_NET, D_OUT_NODE), lambda i: (0, 0)),
          pl.BlockSpec((D_NET, D_OUT_NODE), lambda i: (0, 0)),
      ],
      out_specs=pl.BlockSpec((_M_BLK, D_OUT_NODE), lambda i: (i, 0)),
      out_shape=jax.ShapeDtypeStruct((E_PAD, D_OUT_NODE), jnp.float32),
  )(pin_feat, srcnet, rmat, smat, t2, bmat)


def _net_body(a0_ref, a1_ref, c0_ref, c1_ref, b_ref, o_ref):
  agg = a0_ref[...] + a1_ref[...]
  deg = jnp.maximum(c0_ref[...][:, :1] + c1_ref[...][:, :1], 1.0)
  o_ref[...] = agg * lax.rsqrt(deg) + b_ref[...]


def _net_call(a0, a1, c0, c1, b):
  return pl.pallas_call(
      _net_body,
      out_shape=jax.ShapeDtypeStruct((N_NETS, D_OUT_NET), jnp.float32),
  )(a0, a1, c0, c1, b)


def _node_body(n0_ref, n1_ref, c0_ref, c1_ref, b_ref, o_ref):
  acc = n0_ref[...] + n1_ref[...]
  deg = jnp.maximum(c0_ref[...][:, :1] + c1_ref[...][:, :1], 1.0)
  o_ref[...] = acc / deg + b_ref[...]


def _node_call(n0, n1, c0, c1, b):
  return pl.pallas_call(
      _node_body,
      out_shape=jax.ShapeDtypeStruct((N_NODES, D_OUT_NODE), jnp.float32),
  )(n0, n1, c0, c1, b)


_R_NP = np.repeat(np.eye(D_PIN, dtype=np.float32), D_NET, axis=1)
_S_NP = np.tile(np.eye(D_NET, dtype=np.float32), (1, D_PIN))


@jax.jit
def kernel(node_feat, net_feat, pin_feat, pins_src, pins_dst, pinned_src,
           pinned_dst, W_conv, b_conv, W_lin, b_lin, b_nn):
  pins_src = pins_src.astype(jnp.int32)
  pins_dst = pins_dst.astype(jnp.int32)
  pinned_src = pinned_src.astype(jnp.int32)
  pinned_dst = pinned_dst.astype(jnp.int32)
  npad = E_PAD - E

  def padded(idx, fill):
    return jnp.pad(idx, (0, npad), constant_values=fill).reshape(
        E_PAD // CHUNK, CHUNK)

  ps_cnt = padded(pins_src, N_NODES)
  ps_gat = padded(pins_src, 0)
  pd_idx = padded(pins_dst, N_NETS)
  nd_idx = padded(pinned_dst, N_NODES)
  nsrc_idx = padded(pinned_src, 0)

  ones16 = jnp.ones((CHUNK, 16), jnp.float32)
  zc = jnp.zeros((N_NODES_P, 16), jnp.float32)

  cnt_src, cnt_dst, cnt_in, srcnet = _sc1(
      ps_cnt, pd_idx, nd_idx, nsrc_idx, net_feat, ones16, zc)

  h = _h_call(node_feat, cnt_src[0, :N_NODES], cnt_src[1, :N_NODES], W_conv)

  t2 = W_lin.reshape(D_PIN * D_NET, D_OUT_NODE)
  bmat = b_lin.reshape(D_NET, D_OUT_NODE)
  pin_pad = jnp.pad(pin_feat, ((0, npad), (0, 0)))
  m = _m_call(pin_pad, srcnet, jnp.asarray(_R_NP), jnp.asarray(_S_NP), t2,
              bmat)

  zbig = jnp.zeros((N_NETS_P, D_OUT_NET), jnp.float32)
  zsmall = jnp.zeros((N_NODES_P, D_OUT_NODE), jnp.float32)
  agg, nacc = _sc2(h, m, ps_gat, pd_idx, nd_idx, zbig, zsmall)

  net_out = _net_call(agg[0, :N_NETS], agg[1, :N_NETS],
                      cnt_dst[0, :N_NETS], cnt_dst[1, :N_NETS],
                      b_conv.reshape(1, D_OUT_NET))
  node_out = _node_call(nacc[0, :N_NODES], nacc[1, :N_NODES],
                        cnt_in[0, :N_NODES], cnt_in[1, :N_NODES],
                        b_nn.reshape(1, D_OUT_NODE))
  return (node_out, net_out)
